# baseline (speedup 1.0000x reference)
import functools

import jax
import jax.numpy as jnp
from jax.experimental import pallas as pl
from jax.experimental.pallas import tpu as pltpu


def _gi_body(x_ref, wih_t_ref, bih_ref, out_ref):
    out_ref[...] = (
        jnp.dot(x_ref[...], wih_t_ref[...], preferred_element_type=jnp.float32)
        + bih_ref[...]
    )


def _gi_matmul(x, w_ih, b_ih, block_s):
    S, D = x.shape
    G = w_ih.shape[0]
    grid = (S // block_s,)
    return pl.pallas_call(
        _gi_body,
        grid=grid,
        in_specs=[
            pl.BlockSpec((block_s, D), lambda i: (i, 0)),
            pl.BlockSpec((D, G), lambda i: (0, 0)),
            pl.BlockSpec((1, G), lambda i: (0, 0)),
        ],
        out_specs=pl.BlockSpec((block_s, G), lambda i: (i, 0)),
        out_shape=jax.ShapeDtypeStruct((S, G), jnp.float32),
    )(x, w_ih.T, b_ih.reshape(1, G))


_CHUNK = 8


def _scan_body(gi_ref, whh_t_ref, bhh_ref, wrt_ref, brt_ref,
               hs_ref, ht_ref, logits_ref, h_scr, *, hr):
    @pl.when(pl.program_id(0) == 0)
    def _():
        h_scr[...] = jnp.zeros_like(h_scr)

    h = h_scr[...]
    whh = whh_t_ref[...]
    bhh = bhh_ref[...]
    for i in range(_CHUNK):
        gi = gi_ref[i:i + 1, :]
        h_col = h.T
        gh = jnp.sum(whh * h_col, axis=0, keepdims=True) + bhh
        ir = gi[:, 0:hr]
        iz = gi[:, hr:2 * hr]
        inn = gi[:, 2 * hr:3 * hr]
        hr_g = gh[:, 0:hr]
        hz = gh[:, hr:2 * hr]
        hn = gh[:, 2 * hr:3 * hr]
        r = jax.nn.sigmoid(ir + hr_g)
        z = jax.nn.sigmoid(iz + hz)
        n = jnp.tanh(inn + r * hn)
        h = (1.0 - z) * n + z * h
        hs_ref[i:i + 1, :] = h
    h_scr[...] = h
    ht_ref[...] = h
    logits_ref[...] = (
        jnp.dot(hs_ref[...], wrt_ref[...], preferred_element_type=jnp.float32)
        + brt_ref[...]
    )


def _gru_scan(gi, w_hh, b_hh, w_route, b_route):
    S, G = gi.shape
    HR = w_hh.shape[1]
    E = w_route.shape[0]
    return pl.pallas_call(
        functools.partial(_scan_body, hr=HR),
        grid=(S // _CHUNK,),
        in_specs=[
            pl.BlockSpec((_CHUNK, G), lambda t: (t, 0)),
            pl.BlockSpec((HR, G), lambda t: (0, 0)),
            pl.BlockSpec((1, G), lambda t: (0, 0)),
            pl.BlockSpec((HR, E), lambda t: (0, 0)),
            pl.BlockSpec((1, E), lambda t: (0, 0)),
        ],
        out_specs=(
            pl.BlockSpec((_CHUNK, HR), lambda t: (t, 0)),
            pl.BlockSpec((1, HR), lambda t: (0, 0)),
            pl.BlockSpec((_CHUNK, E), lambda t: (t, 0)),
        ),
        out_shape=(
            jax.ShapeDtypeStruct((S, HR), jnp.float32),
            jax.ShapeDtypeStruct((1, HR), jnp.float32),
            jax.ShapeDtypeStruct((S, E), jnp.float32),
        ),
        scratch_shapes=[pltpu.VMEM((1, HR), jnp.float32)],
    )(gi, w_hh.T, b_hh.reshape(1, G), w_route.T, b_route.reshape(1, E))


def _routing_body(logits_ref, combine_ref, aux_ref, *, n_exp):
    logits = logits_ref[...]
    S = logits.shape[0]
    m = jnp.max(logits, axis=1, keepdims=True)
    ex = jnp.exp(logits - m)
    p = ex / jnp.sum(ex, axis=1, keepdims=True)

    col = jax.lax.broadcasted_iota(jnp.int32, p.shape, 1)
    big = jnp.int32(n_exp + 1)
    m1 = jnp.max(p, axis=1, keepdims=True)
    i1 = jnp.min(jnp.where(p == m1, col, big), axis=1, keepdims=True)
    p2 = jnp.where(col == i1, -jnp.inf, p)
    m2 = jnp.max(p2, axis=1, keepdims=True)
    i2 = jnp.min(jnp.where(p2 == m2, col, big), axis=1, keepdims=True)

    denom = m1 + m2
    w1 = m1 / denom
    w2 = m2 / denom
    combine_ref[...] = jnp.where(col == i1, w1, 0.0) + jnp.where(col == i2, w2, 0.0)

    onehot1 = jnp.where(col == i1, 1.0, 0.0)
    f = jnp.sum(onehot1, axis=0, keepdims=True) / jnp.float32(S)
    P = jnp.sum(p, axis=0, keepdims=True) / jnp.float32(S)
    aux_ref[...] = jnp.float32(n_exp) * jnp.sum(f * P, axis=1, keepdims=True)


def _routing(logits):
    S, E = logits.shape
    return pl.pallas_call(
        functools.partial(_routing_body, n_exp=E),
        out_shape=(
            jax.ShapeDtypeStruct((S, E), jnp.float32),
            jax.ShapeDtypeStruct((1, 1), jnp.float32),
        ),
    )(logits)


def _ffn_body(x_ref, wg_ref, wu_ref, wd_ref, comb_ref, out_ref):
    e = pl.program_id(0)
    h = pl.program_id(1)
    x = x_ref[...]
    wg = wg_ref[0].astype(jnp.bfloat16)
    wu = wu_ref[0].astype(jnp.bfloat16)
    wd = wd_ref[0].astype(jnp.bfloat16)
    dn = (((1,), (1,)), ((), ()))
    gate = jax.lax.dot_general(x, wg, dn, preferred_element_type=jnp.float32)
    up = jax.lax.dot_general(x, wu, dn, preferred_element_type=jnp.float32)
    gelu = 0.5 * gate * (1.0 + jax.lax.erf(gate * 0.7071067811865476))
    hid = (gelu * up).astype(wd.dtype)
    y = jax.lax.dot_general(hid, wd, dn, preferred_element_type=jnp.float32)
    c = comb_ref[0]
    contrib = y * c
    first = jnp.logical_and(e == 0, h == 0)

    @pl.when(first)
    def _():
        out_ref[...] = contrib

    @pl.when(jnp.logical_not(first))
    def _():
        out_ref[...] += contrib


def _dense_ffn(x, wg, wu, wd, combine, block_h):
    S, D = x.shape
    E, H, _ = wg.shape
    comb_t = combine.T.reshape(E, S, 1)
    grid = (E, H // block_h)
    return pl.pallas_call(
        _ffn_body,
        grid=grid,
        in_specs=[
            pl.BlockSpec((S, D), lambda e, h: (0, 0)),
            pl.BlockSpec((1, block_h, D), lambda e, h: (e, h, 0)),
            pl.BlockSpec((1, block_h, D), lambda e, h: (e, h, 0)),
            pl.BlockSpec((1, D, block_h), lambda e, h: (e, 0, h)),
            pl.BlockSpec((1, S, 1), lambda e, h: (e, 0, 0)),
        ],
        out_specs=pl.BlockSpec((S, D), lambda e, h: (0, 0)),
        out_shape=jax.ShapeDtypeStruct((S, D), jnp.float32),
    )(x, wg, wu, wd, comb_t)


def kernel(hidden_states, W_ih, W_hh, b_ih, b_hh, W_route, b_route, Wg, Wu, Wd):
    B, S, D = hidden_states.shape
    x = hidden_states.reshape(S, D)
    block_s = 256 if S % 256 == 0 else S
    block_h = 512 if Wg.shape[1] % 512 == 0 else Wg.shape[1]

    gi = _gi_matmul(x, W_ih, b_ih, block_s)
    hs, ht, logits = _gru_scan(gi, W_hh, b_hh, W_route, b_route)
    combine, aux = _routing(logits)
    final = _dense_ffn(x.astype(jnp.bfloat16), Wg, Wu, Wd, combine, block_h)
    return final.reshape(B, S, D), ht, aux[0, 0]


# speedup vs baseline: 1.3590x; 1.3590x over previous
import functools

import jax
import jax.numpy as jnp
from jax.experimental import pallas as pl
from jax.experimental.pallas import tpu as pltpu
from jax.experimental.pallas import tpu_sc as plsc


def _gi_body(x_ref, wih_t_ref, bih_ref, out_ref):
    out_ref[...] = (
        jnp.dot(x_ref[...], wih_t_ref[...], preferred_element_type=jnp.float32)
        + bih_ref[...]
    )


def _gi_matmul(x, w_ih, b_ih, block_s):
    S, D = x.shape
    G = w_ih.shape[0]
    grid = (S // block_s,)
    return pl.pallas_call(
        _gi_body,
        grid=grid,
        in_specs=[
            pl.BlockSpec((block_s, D), lambda i: (i, 0)),
            pl.BlockSpec((D, G), lambda i: (0, 0)),
            pl.BlockSpec((1, G), lambda i: (0, 0)),
        ],
        out_specs=pl.BlockSpec((block_s, G), lambda i: (i, 0)),
        out_shape=jax.ShapeDtypeStruct((S, G), jnp.float32),
    )(x, w_ih.T, b_ih.reshape(1, G))


_CHUNK = 8


def _scan_body(gi_ref, whh_t_ref, bhh_ref, wrt_ref, brt_ref,
               hs_ref, ht_ref, logits_ref, h_scr, *, hr):
    @pl.when(pl.program_id(0) == 0)
    def _():
        h_scr[...] = jnp.zeros_like(h_scr)

    h = h_scr[...]
    whh = whh_t_ref[...]
    bhh = bhh_ref[...]
    for i in range(_CHUNK):
        gi = gi_ref[i:i + 1, :]
        gh = jnp.dot(h, whh, preferred_element_type=jnp.float32) + bhh
        ir = gi[:, 0:hr]
        iz = gi[:, hr:2 * hr]
        inn = gi[:, 2 * hr:3 * hr]
        hr_g = gh[:, 0:hr]
        hz = gh[:, hr:2 * hr]
        hn = gh[:, 2 * hr:3 * hr]
        r = jax.nn.sigmoid(ir + hr_g)
        z = jax.nn.sigmoid(iz + hz)
        n = jnp.tanh(inn + r * hn)
        h = (1.0 - z) * n + z * h
        hs_ref[i:i + 1, :] = h
    h_scr[...] = h
    ht_ref[...] = h
    logits_ref[...] = (
        jnp.dot(hs_ref[...], wrt_ref[...], preferred_element_type=jnp.float32)
        + brt_ref[...]
    )


def _gru_scan(gi, w_hh, b_hh, w_route, b_route):
    S, G = gi.shape
    HR = w_hh.shape[1]
    E = w_route.shape[0]
    return pl.pallas_call(
        functools.partial(_scan_body, hr=HR),
        grid=(S // _CHUNK,),
        in_specs=[
            pl.BlockSpec((_CHUNK, G), lambda t: (t, 0)),
            pl.BlockSpec((HR, G), lambda t: (0, 0)),
            pl.BlockSpec((1, G), lambda t: (0, 0)),
            pl.BlockSpec((HR, E), lambda t: (0, 0)),
            pl.BlockSpec((1, E), lambda t: (0, 0)),
        ],
        out_specs=(
            pl.BlockSpec((_CHUNK, HR), lambda t: (t, 0)),
            pl.BlockSpec((1, HR), lambda t: (0, 0)),
            pl.BlockSpec((_CHUNK, E), lambda t: (t, 0)),
        ),
        out_shape=(
            jax.ShapeDtypeStruct((S, HR), jnp.float32),
            jax.ShapeDtypeStruct((1, HR), jnp.float32),
            jax.ShapeDtypeStruct((S, E), jnp.float32),
        ),
        scratch_shapes=[pltpu.VMEM((1, HR), jnp.float32)],
    )(gi, w_hh.T, b_hh.reshape(1, G), w_route.T, b_route.reshape(1, E))


def _routing_body(logits_ref, combine_ref, aux_ref, *, n_exp):
    logits = logits_ref[...]
    S = logits.shape[0]
    m = jnp.max(logits, axis=1, keepdims=True)
    ex = jnp.exp(logits - m)
    p = ex / jnp.sum(ex, axis=1, keepdims=True)

    col = jax.lax.broadcasted_iota(jnp.int32, p.shape, 1)
    big = jnp.int32(n_exp + 1)
    m1 = jnp.max(p, axis=1, keepdims=True)
    i1 = jnp.min(jnp.where(p == m1, col, big), axis=1, keepdims=True)
    p2 = jnp.where(col == i1, -jnp.inf, p)
    m2 = jnp.max(p2, axis=1, keepdims=True)
    i2 = jnp.min(jnp.where(p2 == m2, col, big), axis=1, keepdims=True)

    denom = m1 + m2
    w1 = m1 / denom
    w2 = m2 / denom
    combine_ref[...] = jnp.where(col == i1, w1, 0.0) + jnp.where(col == i2, w2, 0.0)

    onehot1 = jnp.where(col == i1, 1.0, 0.0)
    f = jnp.sum(onehot1, axis=0, keepdims=True) / jnp.float32(S)
    P = jnp.sum(p, axis=0, keepdims=True) / jnp.float32(S)
    aux_ref[...] = jnp.float32(n_exp) * jnp.sum(f * P, axis=1, keepdims=True)


def _routing(logits):
    S, E = logits.shape
    return pl.pallas_call(
        functools.partial(_routing_body, n_exp=E),
        out_shape=(
            jax.ShapeDtypeStruct((S, E), jnp.float32),
            jax.ShapeDtypeStruct((1, 1), jnp.float32),
        ),
    )(logits)


_BM = 512
_NBMAX = 16


def _route_dispatch_body(logits_ref, w1_ref, w2_ref, pos1_ref, pos2_ref,
                         meta_ref, aux_ref, *, n_exp, bm, nbmax):
    logits = logits_ref[...]
    S = logits.shape[0]
    m = jnp.max(logits, axis=1, keepdims=True)
    ex = jnp.exp(logits - m)
    p = ex / jnp.sum(ex, axis=1, keepdims=True)

    col = jax.lax.broadcasted_iota(jnp.int32, p.shape, 1)
    big = jnp.int32(n_exp + 1)
    m1 = jnp.max(p, axis=1, keepdims=True)
    i1 = jnp.min(jnp.where(p == m1, col, big), axis=1, keepdims=True)
    p2 = jnp.where(col == i1, -jnp.inf, p)
    m2 = jnp.max(p2, axis=1, keepdims=True)
    i2 = jnp.min(jnp.where(p2 == m2, col, big), axis=1, keepdims=True)

    denom = m1 + m2
    w1_ref[...] = m1 / denom
    w2_ref[...] = m2 / denom

    M1 = jnp.where(col == i1, 1.0, 0.0)
    M2 = jnp.where(col == i2, 1.0, 0.0)
    f = jnp.sum(M1, axis=0, keepdims=True) / jnp.float32(S)
    P = jnp.sum(p, axis=0, keepdims=True) / jnp.float32(S)
    aux_ref[...] = jnp.float32(n_exp) * jnp.sum(f * P, axis=1, keepdims=True)

    M = M1 + M2
    blk = 256
    riota = jax.lax.broadcasted_iota(jnp.int32, (blk, blk), 0)
    ciota = jax.lax.broadcasted_iota(jnp.int32, (blk, blk), 1)
    ltri = jnp.where(riota > ciota, 1.0, 0.0)
    carry = jnp.zeros((1, n_exp), jnp.float32)
    c_parts = []
    for b in range(S // blk):
        Mb = M[b * blk:(b + 1) * blk, :]
        Cb = jnp.dot(ltri, Mb, preferred_element_type=jnp.float32) + carry
        c_parts.append(Cb)
        carry = carry + jnp.sum(Mb, axis=0, keepdims=True)
    C = jnp.concatenate(c_parts, axis=0)
    cnt = carry

    pcnt = jnp.ceil(cnt * (1.0 / bm)) * bm
    acc = jnp.zeros((1, 1), jnp.float32)
    off_parts = []
    for e in range(n_exp):
        off_parts.append(acc)
        acc = acc + pcnt[:, e:e + 1]
    offs = jnp.concatenate(off_parts, axis=1)

    pos1 = jnp.sum(M1 * (offs + C), axis=1, keepdims=True)
    pos2 = jnp.sum(M2 * (offs + C + M1), axis=1, keepdims=True)
    pos1_ref[...] = pos1.astype(jnp.int32)
    pos2_ref[...] = pos2.astype(jnp.int32)

    nlane = meta_ref.shape[1]
    nb = jax.lax.broadcasted_iota(jnp.int32, (1, nlane), 1).astype(jnp.float32)
    ones = jnp.ones((1, nlane), jnp.float32)
    b2e = jnp.zeros((1, nlane), jnp.float32)
    for e in range(n_exp):
        bs = (offs[:, e:e + 1] * (1.0 / bm)) * ones
        be = ((offs[:, e:e + 1] + pcnt[:, e:e + 1]) * (1.0 / bm)) * ones
        b2e = b2e + jnp.float32(e) * jnp.where(
            jnp.logical_and(nb >= bs, nb < be), 1.0, 0.0)
    nact = (acc * (1.0 / bm)) * ones
    sel = jnp.where(nb == jnp.float32(nbmax), nact, b2e)
    meta_ref[...] = sel.astype(jnp.int32)


def _route_dispatch(logits):
    S, E = logits.shape
    return pl.pallas_call(
        functools.partial(_route_dispatch_body, n_exp=E, bm=_BM, nbmax=_NBMAX),
        out_shape=(
            jax.ShapeDtypeStruct((S, 1), jnp.float32),
            jax.ShapeDtypeStruct((S, 1), jnp.float32),
            jax.ShapeDtypeStruct((S, 1), jnp.int32),
            jax.ShapeDtypeStruct((S, 1), jnp.int32),
            jax.ShapeDtypeStruct((1, 128), jnp.int32),
            jax.ShapeDtypeStruct((1, 1), jnp.float32),
        ),
    )(logits)


def _gffn_body(b2e_ref, nact_ref, gx_ref, wg_ref, wu_ref, wd_ref, gy_ref):
    nb = pl.program_id(0)
    h = pl.program_id(1)

    @pl.when(nb < nact_ref[0])
    def _():
        x = gx_ref[...].astype(jnp.bfloat16)
        wg = wg_ref[0].astype(jnp.bfloat16)
        wu = wu_ref[0].astype(jnp.bfloat16)
        wd = wd_ref[0].astype(jnp.bfloat16)
        dn = (((1,), (1,)), ((), ()))
        gate = jax.lax.dot_general(x, wg, dn, preferred_element_type=jnp.float32)
        up = jax.lax.dot_general(x, wu, dn, preferred_element_type=jnp.float32)
        gelu = 0.5 * gate * (1.0 + jax.lax.erf(gate * 0.7071067811865476))
        hid = (gelu * up).astype(jnp.bfloat16)
        y = jax.lax.dot_general(hid, wd, dn, preferred_element_type=jnp.float32)

        @pl.when(h == 0)
        def _():
            gy_ref[...] = y

        @pl.when(h != 0)
        def _():
            gy_ref[...] += y


def _grouped_ffn(gx, wg, wu, wd, b2e, nact, block_h):
    TP, D = gx.shape
    E, H, _ = wg.shape
    grid = (_NBMAX, H // block_h)
    grid_spec = pltpu.PrefetchScalarGridSpec(
        num_scalar_prefetch=2,
        grid=grid,
        in_specs=[
            pl.BlockSpec((_BM, D), lambda nb, h, b2e, nact: (nb, 0)),
            pl.BlockSpec((1, block_h, D),
                         lambda nb, h, b2e, nact: (b2e[nb], h, 0)),
            pl.BlockSpec((1, block_h, D),
                         lambda nb, h, b2e, nact: (b2e[nb], h, 0)),
            pl.BlockSpec((1, D, block_h),
                         lambda nb, h, b2e, nact: (b2e[nb], 0, h)),
        ],
        out_specs=pl.BlockSpec((_BM, D), lambda nb, h, b2e, nact: (nb, 0)),
    )
    return pl.pallas_call(
        _gffn_body,
        grid_spec=grid_spec,
        out_shape=jax.ShapeDtypeStruct((TP, D), jnp.float32),
    )(b2e, nact, gx, wg, wu, wd)


def _sc_scatter(x, pos1, pos2, tot_pad):
    S, D = x.shape
    info = plsc.get_sparse_core_info()
    nw = info.num_cores * info.num_subcores
    tw = S // nw
    mesh = plsc.VectorSubcoreMesh(core_axis_name="c", subcore_axis_name="s")

    @functools.partial(
        pl.kernel, mesh=mesh,
        out_type=jax.ShapeDtypeStruct((tot_pad, D), jnp.float32),
        scratch_types=[
            pltpu.VMEM((tw,), jnp.int32),
            pltpu.VMEM((tw,), jnp.int32),
            pltpu.VMEM((tw, D), jnp.float32),
            pltpu.SemaphoreType.DMA,
        ],
    )
    def k(x_hbm, p1_hbm, p2_hbm, gx_hbm, i1_v, i2_v, rows_v, sem):
        wid = jax.lax.axis_index("s") * info.num_cores + jax.lax.axis_index("c")
        base = wid * tw
        pltpu.sync_copy(p1_hbm.at[pl.ds(base, tw)], i1_v)
        pltpu.sync_copy(p2_hbm.at[pl.ds(base, tw)], i2_v)
        pltpu.sync_copy(x_hbm.at[pl.ds(base, tw)], rows_v)
        pltpu.async_copy(rows_v, gx_hbm.at[i1_v], sem).wait()
        pltpu.async_copy(rows_v, gx_hbm.at[i2_v], sem).wait()

    return k(x, pos1, pos2)


def _sc_gather(gy, pos1, pos2):
    TP, D = gy.shape
    S = pos1.shape[0]
    info = plsc.get_sparse_core_info()
    nw = info.num_cores * info.num_subcores
    tw = S // nw
    mesh = plsc.VectorSubcoreMesh(core_axis_name="c", subcore_axis_name="s")

    @functools.partial(
        pl.kernel, mesh=mesh,
        out_type=(
            jax.ShapeDtypeStruct((S, D), jnp.float32),
            jax.ShapeDtypeStruct((S, D), jnp.float32),
        ),
        scratch_types=[
            pltpu.VMEM((tw,), jnp.int32),
            pltpu.VMEM((tw, D), jnp.float32),
            pltpu.SemaphoreType.DMA,
        ],
    )
    def k(gy_hbm, p1_hbm, p2_hbm, y1_hbm, y2_hbm, idx_v, rows_v, sem):
        wid = jax.lax.axis_index("s") * info.num_cores + jax.lax.axis_index("c")
        base = wid * tw
        pltpu.sync_copy(p1_hbm.at[pl.ds(base, tw)], idx_v)
        pltpu.async_copy(gy_hbm.at[idx_v], rows_v, sem).wait()
        pltpu.sync_copy(rows_v, y1_hbm.at[pl.ds(base, tw)])
        pltpu.sync_copy(p2_hbm.at[pl.ds(base, tw)], idx_v)
        pltpu.async_copy(gy_hbm.at[idx_v], rows_v, sem).wait()
        pltpu.sync_copy(rows_v, y2_hbm.at[pl.ds(base, tw)])

    return k(gy, pos1, pos2)


def _wadd_body(y1_ref, y2_ref, w1_ref, w2_ref, out_ref):
    out_ref[...] = w1_ref[...] * y1_ref[...] + w2_ref[...] * y2_ref[...]


def _weighted_add(y1, y2, w1, w2, block_s):
    S, D = y1.shape
    return pl.pallas_call(
        _wadd_body,
        grid=(S // block_s,),
        in_specs=[
            pl.BlockSpec((block_s, D), lambda i: (i, 0)),
            pl.BlockSpec((block_s, D), lambda i: (i, 0)),
            pl.BlockSpec((block_s, 1), lambda i: (i, 0)),
            pl.BlockSpec((block_s, 1), lambda i: (i, 0)),
        ],
        out_specs=pl.BlockSpec((block_s, D), lambda i: (i, 0)),
        out_shape=jax.ShapeDtypeStruct((S, D), jnp.float32),
    )(y1, y2, w1, w2)


def _ffn_body(x_ref, wg_ref, wu_ref, wd_ref, comb_ref, out_ref):
    e = pl.program_id(0)
    h = pl.program_id(1)
    x = x_ref[...]
    wg = wg_ref[0].astype(jnp.bfloat16)
    wu = wu_ref[0].astype(jnp.bfloat16)
    wd = wd_ref[0].astype(jnp.bfloat16)
    dn = (((1,), (1,)), ((), ()))
    gate = jax.lax.dot_general(x, wg, dn, preferred_element_type=jnp.float32)
    up = jax.lax.dot_general(x, wu, dn, preferred_element_type=jnp.float32)
    gelu = 0.5 * gate * (1.0 + jax.lax.erf(gate * 0.7071067811865476))
    hid = (gelu * up).astype(wd.dtype)
    y = jax.lax.dot_general(hid, wd, dn, preferred_element_type=jnp.float32)
    c = comb_ref[0]
    contrib = y * c
    first = jnp.logical_and(e == 0, h == 0)

    @pl.when(first)
    def _():
        out_ref[...] = contrib

    @pl.when(jnp.logical_not(first))
    def _():
        out_ref[...] += contrib


def _dense_ffn(x, wg, wu, wd, combine, block_h):
    S, D = x.shape
    E, H, _ = wg.shape
    comb_t = combine.T.reshape(E, S, 1)
    grid = (E, H // block_h)
    return pl.pallas_call(
        _ffn_body,
        grid=grid,
        in_specs=[
            pl.BlockSpec((S, D), lambda e, h: (0, 0)),
            pl.BlockSpec((1, block_h, D), lambda e, h: (e, h, 0)),
            pl.BlockSpec((1, block_h, D), lambda e, h: (e, h, 0)),
            pl.BlockSpec((1, D, block_h), lambda e, h: (e, 0, h)),
            pl.BlockSpec((1, S, 1), lambda e, h: (e, 0, 0)),
        ],
        out_specs=pl.BlockSpec((S, D), lambda e, h: (0, 0)),
        out_shape=jax.ShapeDtypeStruct((S, D), jnp.float32),
    )(x, wg, wu, wd, comb_t)


def kernel(hidden_states, W_ih, W_hh, b_ih, b_hh, W_route, b_route, Wg, Wu, Wd):
    B, S, D = hidden_states.shape
    x = hidden_states.reshape(S, D)
    block_s = 256 if S % 256 == 0 else S
    block_h = 512 if Wg.shape[1] % 512 == 0 else Wg.shape[1]

    gi = _gi_matmul(x, W_ih, b_ih, block_s)
    hs, ht, logits = _gru_scan(gi, W_hh, b_hh, W_route, b_route)
    w1, w2, pos1, pos2, meta, aux = _route_dispatch(logits)
    b2e = meta[0, :_NBMAX]
    nact = meta[0, _NBMAX:_NBMAX + 1]
    tot_pad = _NBMAX * _BM
    gx = _sc_scatter(x, pos1.reshape(S), pos2.reshape(S), tot_pad)
    gy = _grouped_ffn(gx, Wg, Wu, Wd, b2e, nact, block_h)
    y1, y2 = _sc_gather(gy, pos1.reshape(S), pos2.reshape(S))
    final = _weighted_add(y1, y2, w1, w2, block_s)
    return final.reshape(B, S, D), ht, aux[0, 0]
